# baseline (speedup 1.0000x reference)
import functools
import jax
import jax.numpy as jnp
from jax.experimental import pallas as pl
from jax.experimental.pallas import tpu as pltpu
from jax.experimental.pallas import tpu_sc as plsc

B, N, D_IN, D_OUT, D_HID = 4, 4096, 128, 256, 64
K = 32


def _prologue_body(f_ref, p_ref, wpre_ref, bpre_ref, wsc_ref, bsc_ref,
                   wpab_ref, bp2f_ref, tc_ref, res_ref):
    f = f_ref[0]
    p = p_ref[0]
    h = f @ wpre_ref[...] + bpre_ref[...]
    h = jnp.maximum(h, 0.01 * h)
    pa = p @ wpab_ref[...]
    t = h + pa[:, D_HID:]
    c = pa[:, :D_HID] - h + bp2f_ref[...]
    tc_ref[0] = jnp.concatenate([t, c], axis=1)
    res_ref[0] = f @ wsc_ref[...] + bsc_ref[...]


def _prologue(f, p, W_pre, b_pre, W_sc, b_sc, W_pab, b_p2f):
    T = 1024
    grid = (B, N // T)
    return pl.pallas_call(
        _prologue_body,
        grid=grid,
        in_specs=[
            pl.BlockSpec((1, T, D_IN), lambda b, i: (b, i, 0)),
            pl.BlockSpec((1, T, 3), lambda b, i: (b, i, 0)),
            pl.BlockSpec((D_IN, D_HID), lambda b, i: (0, 0)),
            pl.BlockSpec((D_HID,), lambda b, i: (0,)),
            pl.BlockSpec((D_IN, D_OUT), lambda b, i: (0, 0)),
            pl.BlockSpec((D_OUT,), lambda b, i: (0,)),
            pl.BlockSpec((3, 2 * D_HID), lambda b, i: (0, 0)),
            pl.BlockSpec((D_HID,), lambda b, i: (0,)),
        ],
        out_specs=[
            pl.BlockSpec((1, T, 2 * D_HID), lambda b, i: (b, i, 0)),
            pl.BlockSpec((1, T, D_OUT), lambda b, i: (b, i, 0)),
        ],
        out_shape=[
            jax.ShapeDtypeStruct((B, N, 2 * D_HID), jnp.float32),
            jax.ShapeDtypeStruct((B, N, D_OUT), jnp.float32),
        ],
    )(f, p, W_pre, b_pre, W_sc, b_sc, W_pab, b_p2f)


def _tail_body(tg_ref, c_ref, res_ref, wmlp_ref, bmlp_ref, wpst_ref,
               bpst_ref, out_ref):
    Trows = c_ref.shape[1]
    tg = tg_ref[0, :, :, :D_HID]
    c = c_ref[0, :, D_HID:]
    g = tg + c[None, :, :]
    g = jnp.maximum(g, 0.01 * g)
    z = g.reshape(K * Trows, D_HID) @ wmlp_ref[...] + bmlp_ref[...]
    z = jnp.maximum(z, 0.01 * z)
    m = jnp.max(z.reshape(K, Trows, D_HID), axis=0)
    o = m @ wpst_ref[...] + bpst_ref[...] + res_ref[0]
    out_ref[0] = jnp.maximum(o, 0.01 * o)


def _tail(tg, tc_pack, res, W_mlp, b_mlp, W_pst, b_pst):
    T = 512
    grid = (B, N // T)
    return pl.pallas_call(
        _tail_body,
        grid=grid,
        in_specs=[
            pl.BlockSpec((1, K, T, 2 * D_HID), lambda b, i: (b, 0, i, 0)),
            pl.BlockSpec((1, T, 2 * D_HID), lambda b, i: (b, i, 0)),
            pl.BlockSpec((1, T, D_OUT), lambda b, i: (b, i, 0)),
            pl.BlockSpec((D_HID, D_HID), lambda b, i: (0, 0)),
            pl.BlockSpec((D_HID,), lambda b, i: (0,)),
            pl.BlockSpec((D_HID, D_OUT), lambda b, i: (0, 0)),
            pl.BlockSpec((D_OUT,), lambda b, i: (0,)),
        ],
        out_specs=pl.BlockSpec((1, T, D_OUT), lambda b, i: (b, i, 0)),
        out_shape=jax.ShapeDtypeStruct((B, N, D_OUT), jnp.float32),
    )(tg, tc_pack, res, W_mlp, b_mlp, W_pst, b_pst)


def _rev0(a):
    if a.shape[0] == 1:
        return a
    return jnp.concatenate([a[i:i + 1] for i in range(a.shape[0] - 1, -1, -1)],
                           axis=0)


def _cmpex(ka, xa, kb, xb):
    m = kb < ka
    klo = jnp.where(m, kb, ka)
    khi = jnp.where(m, ka, kb)
    xlo = jnp.where(m, xb, xa)
    xhi = jnp.where(m, xa, xb)
    return klo, xlo, khi, xhi


def _bitonic_clean(k, x):
    R = k.shape[0]
    d = R // 2
    while d >= 1:
        kparts, xparts = [], []
        for b in range(0, R, 2 * d):
            ka, kb = k[b:b + d], k[b + d:b + 2 * d]
            if x is None:
                kparts += [jnp.minimum(ka, kb), jnp.maximum(ka, kb)]
            else:
                klo, xlo, khi, xhi = _cmpex(ka, x[b:b + d], kb,
                                            x[b + d:b + 2 * d])
                kparts += [klo, khi]
                xparts += [xlo, xhi]
        k = jnp.concatenate(kparts, axis=0)
        if x is not None:
            x = jnp.concatenate(xparts, axis=0)
        d //= 2
    return k, x


def _merge_packed(a, pw, truncate):
    R, L, _ = a.shape
    bit = 1 << pw
    low = bit - 1
    A = a[:, :L // 2]
    Bh = a[:, L // 2:]
    A = (A & ~bit) + (A & low)
    Bh = (Bh & ~bit) + (Bh & low) + 1
    Br = _rev0(Bh)
    if truncate:
        lo, _ = _bitonic_clean(jnp.minimum(A, Br), None)
        return lo
    v = jnp.concatenate([A, Br], axis=0)
    v, _ = _bitonic_clean(v, None)
    return v


def _merge_kv(k, x, truncate):
    R, L, _ = k.shape
    ka, xa = k[:, :L // 2], x[:, :L // 2]
    kb = _rev0(k[:, L // 2:])
    xb = _rev0(x[:, L // 2:])
    if truncate:
        klo, xlo, _, _ = _cmpex(ka, xa, kb, xb)
        return _bitonic_clean(klo, xlo)
    kc = jnp.concatenate([ka, kb], axis=0)
    xc = jnp.concatenate([xa, xb], axis=0)
    return _bitonic_clean(kc, xc)


_UNPACK_L = 16


def _knn_body(pfull_ref, ptile_ref, idx_ref):
    b = pl.program_id(0)
    p_all = pfull_ref[0]
    p_til = ptile_ref[0]
    T = p_til.shape[0]
    sq = jnp.sum(p_all * p_all, axis=1)
    sqt = jnp.sum(p_til * p_til, axis=1)
    dot = jax.lax.dot_general(p_all, p_til, (((1,), (1,)), ((), ())),
                              preferred_element_type=jnp.float32)
    key = (sq[:, None] - 2.0 * dot) + sqt[None, :]
    kb = jax.lax.bitcast_convert_type(key, jnp.int32)
    a = kb ^ ((kb >> 31) & jnp.int32(0x7FFFFFFF))

    a = a.reshape(1, N, T)
    pw = 0
    while a.shape[0] < K:
        a = _merge_packed(a, pw, truncate=False)
        pw += 1
    while a.shape[1] > _UNPACK_L:
        a = _merge_packed(a, pw, truncate=True)
        pw += 1
    L = a.shape[1]
    m = a & jnp.int32((1 << pw) - 1)
    l_iota = jax.lax.broadcasted_iota(jnp.int32, a.shape, 1)
    x = m * L + l_iota + b * N
    k = a
    while k.shape[1] > 1:
        k, x = _merge_kv(k, x, truncate=True)
    idx_ref[0] = x[:, 0, :]


def _knn(p):
    T = 128
    grid = (B, N // T)
    return pl.pallas_call(
        _knn_body,
        grid=grid,
        in_specs=[
            pl.BlockSpec((1, N, 3), lambda b, i: (b, 0, 0)),
            pl.BlockSpec((1, T, 3), lambda b, i: (b, i, 0)),
        ],
        out_specs=pl.BlockSpec((1, K, T), lambda b, i: (b, 0, i)),
        out_shape=jax.ShapeDtypeStruct((B, K, N), jnp.int32),
    )(p, p)


_N_IDX = B * K * N
_SC_W = 32
_PER_W = _N_IDX // _SC_W
_CHUNK = 512
_ROW_W = 2 * D_HID


def _sc_gather_body(t_ref, idx_ref, out_ref, idx_v, rows_v, sem):
    wid = jax.lax.axis_index("c") * 16 + jax.lax.axis_index("s")
    base = wid * _PER_W

    def body(i, carry):
        off = base + i * _CHUNK
        pltpu.sync_copy(idx_ref.at[pl.ds(off, _CHUNK)], idx_v)
        pltpu.async_copy(t_ref.at[idx_v], rows_v, sem).wait()
        pltpu.sync_copy(rows_v, out_ref.at[pl.ds(off, _CHUNK)])
        return carry

    jax.lax.fori_loop(0, _PER_W // _CHUNK, body, 0)


def _sc_gather(t_flat, idx_flat):
    mesh = plsc.VectorSubcoreMesh(core_axis_name="c", subcore_axis_name="s")
    fn = functools.partial(
        pl.kernel,
        mesh=mesh,
        out_type=jax.ShapeDtypeStruct((_N_IDX, _ROW_W), jnp.float32),
        scratch_types=[
            pltpu.VMEM((_CHUNK,), jnp.int32),
            pltpu.VMEM((_CHUNK, _ROW_W), jnp.float32),
            pltpu.SemaphoreType.DMA,
        ],
    )(_sc_gather_body)
    return fn(t_flat, idx_flat)


def kernel(f, p, W_sc, b_sc, W_pre, b_pre, W_p2f, b_p2f, W_mlp, b_mlp,
           W_pst, b_pst):
    A = W_p2f[0:3] - W_p2f[6:9]
    Bm = W_p2f[3:6] + W_p2f[6:9]
    W_pab = jnp.concatenate([A, Bm], axis=1)

    tc_pack, res = _prologue(f, p, W_pre, b_pre, W_sc, b_sc, W_pab, b_p2f)

    idx = _knn(p)

    tg = _sc_gather(tc_pack.reshape(B * N, _ROW_W), idx.reshape(-1))
    tg = tg.reshape(B, K, N, _ROW_W)

    out = _tail(tg, tc_pack, res, W_mlp, b_mlp, W_pst, b_pst)
    return (out, p)


# speedup vs baseline: 1.2734x; 1.2734x over previous
import functools
import jax
import jax.numpy as jnp
from jax.experimental import pallas as pl
from jax.experimental.pallas import tpu as pltpu
from jax.experimental.pallas import tpu_sc as plsc

B, N, D_IN, D_OUT, D_HID = 4, 4096, 128, 256, 64
K = 32


def _prologue_body(f_ref, p_ref, wpre_ref, bpre_ref, wsc_ref, bsc_ref,
                   wpab_ref, bp2f_ref, tc_ref, res_ref):
    f = f_ref[0]
    p = p_ref[0]
    h = f @ wpre_ref[...] + bpre_ref[...]
    h = jnp.maximum(h, 0.01 * h)
    pa = p @ wpab_ref[...]
    t = h + pa[:, D_HID:]
    c = pa[:, :D_HID] - h + bp2f_ref[...]
    tc_ref[0] = jnp.concatenate([t, c], axis=1)
    res_ref[0] = f @ wsc_ref[...] + bsc_ref[...]


def _prologue(f, p, W_pre, b_pre, W_sc, b_sc, W_pab, b_p2f):
    T = 1024
    grid = (B, N // T)
    return pl.pallas_call(
        _prologue_body,
        grid=grid,
        in_specs=[
            pl.BlockSpec((1, T, D_IN), lambda b, i: (b, i, 0)),
            pl.BlockSpec((1, T, 3), lambda b, i: (b, i, 0)),
            pl.BlockSpec((D_IN, D_HID), lambda b, i: (0, 0)),
            pl.BlockSpec((D_HID,), lambda b, i: (0,)),
            pl.BlockSpec((D_IN, D_OUT), lambda b, i: (0, 0)),
            pl.BlockSpec((D_OUT,), lambda b, i: (0,)),
            pl.BlockSpec((3, 2 * D_HID), lambda b, i: (0, 0)),
            pl.BlockSpec((D_HID,), lambda b, i: (0,)),
        ],
        out_specs=[
            pl.BlockSpec((1, T, 2 * D_HID), lambda b, i: (b, i, 0)),
            pl.BlockSpec((1, T, D_OUT), lambda b, i: (b, i, 0)),
        ],
        out_shape=[
            jax.ShapeDtypeStruct((B, N, 2 * D_HID), jnp.float32),
            jax.ShapeDtypeStruct((B, N, D_OUT), jnp.float32),
        ],
    )(f, p, W_pre, b_pre, W_sc, b_sc, W_pab, b_p2f)


def _tail_body(tg_ref, c_ref, res_ref, wmlp_ref, bmlp_ref, wpst_ref,
               bpst_ref, out_ref):
    Trows = c_ref.shape[1]
    tg = tg_ref[0, :, :, :D_HID]
    c = c_ref[0, :, D_HID:]
    g = tg + c[None, :, :]
    g = jnp.maximum(g, 0.01 * g)
    z = g.reshape(K * Trows, D_HID) @ wmlp_ref[...] + bmlp_ref[...]
    z = jnp.maximum(z, 0.01 * z)
    m = jnp.max(z.reshape(K, Trows, D_HID), axis=0)
    o = m @ wpst_ref[...] + bpst_ref[...] + res_ref[0]
    out_ref[0] = jnp.maximum(o, 0.01 * o)


def _tail(tg, tc_pack, res, W_mlp, b_mlp, W_pst, b_pst):
    T = 512
    grid = (B, N // T)
    return pl.pallas_call(
        _tail_body,
        grid=grid,
        in_specs=[
            pl.BlockSpec((1, K, T, 2 * D_HID), lambda b, i: (b, 0, i, 0)),
            pl.BlockSpec((1, T, 2 * D_HID), lambda b, i: (b, i, 0)),
            pl.BlockSpec((1, T, D_OUT), lambda b, i: (b, i, 0)),
            pl.BlockSpec((D_HID, D_HID), lambda b, i: (0, 0)),
            pl.BlockSpec((D_HID,), lambda b, i: (0,)),
            pl.BlockSpec((D_HID, D_OUT), lambda b, i: (0, 0)),
            pl.BlockSpec((D_OUT,), lambda b, i: (0,)),
        ],
        out_specs=pl.BlockSpec((1, T, D_OUT), lambda b, i: (b, i, 0)),
        out_shape=jax.ShapeDtypeStruct((B, N, D_OUT), jnp.float32),
    )(tg, tc_pack, res, W_mlp, b_mlp, W_pst, b_pst)


def _rev0(a):
    if a.shape[0] == 1:
        return a
    return jnp.concatenate([a[i:i + 1] for i in range(a.shape[0] - 1, -1, -1)],
                           axis=0)


def _cmpex(ka, xa, kb, xb):
    m = kb < ka
    klo = jnp.where(m, kb, ka)
    khi = jnp.where(m, ka, kb)
    xlo = jnp.where(m, xb, xa)
    xhi = jnp.where(m, xa, xb)
    return klo, xlo, khi, xhi


def _bitonic_clean(k, x):
    R = k.shape[0]
    d = R // 2
    while d >= 1:
        kparts, xparts = [], []
        for b in range(0, R, 2 * d):
            ka, kb = k[b:b + d], k[b + d:b + 2 * d]
            if x is None:
                kparts += [jnp.minimum(ka, kb), jnp.maximum(ka, kb)]
            else:
                klo, xlo, khi, xhi = _cmpex(ka, x[b:b + d], kb,
                                            x[b + d:b + 2 * d])
                kparts += [klo, khi]
                xparts += [xlo, xhi]
        k = jnp.concatenate(kparts, axis=0)
        if x is not None:
            x = jnp.concatenate(xparts, axis=0)
        d //= 2
    return k, x


def _repack(h, pw, side):
    if pw == 0:
        if side == 0:
            return h
        hi = jax.lax.bitcast_convert_type(h, jnp.int32) + 1
        return jax.lax.bitcast_convert_type(hi, jnp.float32)
    low = (1 << pw) - 1
    hi = jax.lax.bitcast_convert_type(h, jnp.int32)
    hi = hi + (hi & low) + side
    return jax.lax.bitcast_convert_type(hi, jnp.float32)


def _merge_packed(a, pw, truncate):
    R, L, _ = a.shape
    A = _repack(a[:, :L // 2], pw, 0)
    Br = _rev0(_repack(a[:, L // 2:], pw, 1))
    if truncate:
        lo, _ = _bitonic_clean(jnp.minimum(A, Br), None)
        return lo
    v = jnp.concatenate([A, Br], axis=0)
    v, _ = _bitonic_clean(v, None)
    return v


def _merge_kv(k, x, truncate):
    R, L, _ = k.shape
    ka, xa = k[:, :L // 2], x[:, :L // 2]
    kb = _rev0(k[:, L // 2:])
    xb = _rev0(x[:, L // 2:])
    if truncate:
        klo, xlo, _, _ = _cmpex(ka, xa, kb, xb)
        return _bitonic_clean(klo, xlo)
    kc = jnp.concatenate([ka, kb], axis=0)
    xc = jnp.concatenate([xa, xb], axis=0)
    return _bitonic_clean(kc, xc)


_UNPACK_L = 16


def _knn_body(pfull_ref, ptile_ref, idx_ref):
    b = pl.program_id(0)
    p_all = pfull_ref[0]
    p_til = ptile_ref[0]
    T = p_til.shape[0]
    sq = jnp.sum(p_all * p_all, axis=1)
    sqt = jnp.sum(p_til * p_til, axis=1)
    dot = jax.lax.dot_general(p_all, p_til, (((1,), (1,)), ((), ())),
                              preferred_element_type=jnp.float32)
    key = (sq[:, None] - 2.0 * dot) + sqt[None, :]
    key = jnp.maximum(key, 1e-30)
    kb = jax.lax.bitcast_convert_type(key, jnp.int32)
    kb = kb & jnp.int32(~0xFF)
    a = jax.lax.bitcast_convert_type(kb, jnp.float32)

    a = a.reshape(1, N, T)
    pw = 0
    while a.shape[0] < K:
        a = _merge_packed(a, pw, truncate=False)
        pw += 1
    while a.shape[1] > _UNPACK_L:
        a = _merge_packed(a, pw, truncate=True)
        pw += 1
    L = a.shape[1]
    ai = jax.lax.bitcast_convert_type(a, jnp.int32)
    m = ai & jnp.int32((1 << pw) - 1)
    l_iota = jax.lax.broadcasted_iota(jnp.int32, a.shape, 1)
    x = m * L + l_iota + b * N
    k = a
    while k.shape[1] > 1:
        k, x = _merge_kv(k, x, truncate=True)
    idx_ref[0] = x[:, 0, :]


def _knn(p):
    T = 128
    grid = (B, N // T)
    return pl.pallas_call(
        _knn_body,
        grid=grid,
        in_specs=[
            pl.BlockSpec((1, N, 3), lambda b, i: (b, 0, 0)),
            pl.BlockSpec((1, T, 3), lambda b, i: (b, i, 0)),
        ],
        out_specs=pl.BlockSpec((1, K, T), lambda b, i: (b, 0, i)),
        out_shape=jax.ShapeDtypeStruct((B, K, N), jnp.int32),
    )(p, p)


_N_IDX = B * K * N
_SC_W = 32
_PER_W = _N_IDX // _SC_W
_CHUNK = 512
_ROW_W = 2 * D_HID


def _sc_gather_body(t_ref, idx_ref, out_ref, idx_v, rows_v, sem):
    wid = jax.lax.axis_index("c") * 16 + jax.lax.axis_index("s")
    base = wid * _PER_W

    def body(i, carry):
        off = base + i * _CHUNK
        pltpu.sync_copy(idx_ref.at[pl.ds(off, _CHUNK)], idx_v)
        pltpu.async_copy(t_ref.at[idx_v], rows_v, sem).wait()
        pltpu.sync_copy(rows_v, out_ref.at[pl.ds(off, _CHUNK)])
        return carry

    jax.lax.fori_loop(0, _PER_W // _CHUNK, body, 0)


def _sc_gather(t_flat, idx_flat):
    mesh = plsc.VectorSubcoreMesh(core_axis_name="c", subcore_axis_name="s")
    fn = functools.partial(
        pl.kernel,
        mesh=mesh,
        out_type=jax.ShapeDtypeStruct((_N_IDX, _ROW_W), jnp.float32),
        scratch_types=[
            pltpu.VMEM((_CHUNK,), jnp.int32),
            pltpu.VMEM((_CHUNK, _ROW_W), jnp.float32),
            pltpu.SemaphoreType.DMA,
        ],
    )(_sc_gather_body)
    return fn(t_flat, idx_flat)


def kernel(f, p, W_sc, b_sc, W_pre, b_pre, W_p2f, b_p2f, W_mlp, b_mlp,
           W_pst, b_pst):
    A = W_p2f[0:3] - W_p2f[6:9]
    Bm = W_p2f[3:6] + W_p2f[6:9]
    W_pab = jnp.concatenate([A, Bm], axis=1)

    tc_pack, res = _prologue(f, p, W_pre, b_pre, W_sc, b_sc, W_pab, b_p2f)

    idx = _knn(p)

    tg = _sc_gather(tc_pack.reshape(B * N, _ROW_W), idx.reshape(-1))
    tg = tg.reshape(B, K, N, _ROW_W)

    out = _tail(tg, tc_pack, res, W_mlp, b_mlp, W_pst, b_pst)
    return (out, p)


# speedup vs baseline: 1.4641x; 1.1497x over previous
import functools
import jax
import jax.numpy as jnp
from jax.experimental import pallas as pl
from jax.experimental.pallas import tpu as pltpu
from jax.experimental.pallas import tpu_sc as plsc

B, N, D_IN, D_OUT, D_HID = 4, 4096, 128, 256, 64
K = 32


def _prologue_body(f_ref, p_ref, wpre_ref, bpre_ref, wsc_ref, bsc_ref,
                   wpab_ref, bp2f_ref, tc_ref, res_ref):
    f = f_ref[0]
    p = p_ref[0]
    h = f @ wpre_ref[...] + bpre_ref[...]
    h = jnp.maximum(h, 0.01 * h)
    pa = p @ wpab_ref[...]
    t = h + pa[:, D_HID:]
    c = pa[:, :D_HID] - h + bp2f_ref[...]
    tc_ref[0] = jnp.concatenate([t, c], axis=1)
    res_ref[0] = f @ wsc_ref[...] + bsc_ref[...]


def _prologue(f, p, W_pre, b_pre, W_sc, b_sc, W_pab, b_p2f):
    T = 1024
    grid = (B, N // T)
    return pl.pallas_call(
        _prologue_body,
        grid=grid,
        in_specs=[
            pl.BlockSpec((1, T, D_IN), lambda b, i: (b, i, 0)),
            pl.BlockSpec((1, T, 3), lambda b, i: (b, i, 0)),
            pl.BlockSpec((D_IN, D_HID), lambda b, i: (0, 0)),
            pl.BlockSpec((D_HID,), lambda b, i: (0,)),
            pl.BlockSpec((D_IN, D_OUT), lambda b, i: (0, 0)),
            pl.BlockSpec((D_OUT,), lambda b, i: (0,)),
            pl.BlockSpec((3, 2 * D_HID), lambda b, i: (0, 0)),
            pl.BlockSpec((D_HID,), lambda b, i: (0,)),
        ],
        out_specs=[
            pl.BlockSpec((1, T, 2 * D_HID), lambda b, i: (b, i, 0)),
            pl.BlockSpec((1, T, D_OUT), lambda b, i: (b, i, 0)),
        ],
        out_shape=[
            jax.ShapeDtypeStruct((B, N, 2 * D_HID), jnp.float32),
            jax.ShapeDtypeStruct((B, N, D_OUT), jnp.float32),
        ],
    )(f, p, W_pre, b_pre, W_sc, b_sc, W_pab, b_p2f)


def _tail_body(tg_ref, c_ref, res_ref, wmlp_ref, bmlp_ref, wpst_ref,
               bpst_ref, out_ref):
    Trows = c_ref.shape[0]
    tg = tg_ref[:, :, :D_HID]
    c = c_ref[:, D_HID:]
    g = tg + c[None, :, :]
    g = jnp.maximum(g, 0.01 * g)
    z = g.reshape(K * Trows, D_HID) @ wmlp_ref[...] + bmlp_ref[...]
    z = jnp.maximum(z, 0.01 * z)
    m = jnp.max(z.reshape(K, Trows, D_HID), axis=0)
    o = m @ wpst_ref[...] + bpst_ref[...] + res_ref[...]
    out_ref[...] = jnp.maximum(o, 0.01 * o)


def _tail(tg_b, tc_b, res_b, W_mlp, b_mlp, W_pst, b_pst):
    T = 512
    grid = (N // T,)
    return pl.pallas_call(
        _tail_body,
        grid=grid,
        in_specs=[
            pl.BlockSpec((K, T, 2 * D_HID), lambda i: (0, i, 0)),
            pl.BlockSpec((T, 2 * D_HID), lambda i: (i, 0)),
            pl.BlockSpec((T, D_OUT), lambda i: (i, 0)),
            pl.BlockSpec((D_HID, D_HID), lambda i: (0, 0)),
            pl.BlockSpec((D_HID,), lambda i: (0,)),
            pl.BlockSpec((D_HID, D_OUT), lambda i: (0, 0)),
            pl.BlockSpec((D_OUT,), lambda i: (0,)),
        ],
        out_specs=pl.BlockSpec((T, D_OUT), lambda i: (i, 0)),
        out_shape=jax.ShapeDtypeStruct((N, D_OUT), jnp.float32),
    )(tg_b, tc_b, res_b, W_mlp, b_mlp, W_pst, b_pst)


def _rev0(a):
    if a.shape[0] == 1:
        return a
    return jnp.concatenate([a[i:i + 1] for i in range(a.shape[0] - 1, -1, -1)],
                           axis=0)


def _cmpex(ka, xa, kb, xb):
    m = kb < ka
    klo = jnp.where(m, kb, ka)
    khi = jnp.where(m, ka, kb)
    xlo = jnp.where(m, xb, xa)
    xhi = jnp.where(m, xa, xb)
    return klo, xlo, khi, xhi


def _bitonic_clean(k, x):
    R = k.shape[0]
    d = R // 2
    while d >= 1:
        kparts, xparts = [], []
        for b in range(0, R, 2 * d):
            ka, kb = k[b:b + d], k[b + d:b + 2 * d]
            if x is None:
                kparts += [jnp.minimum(ka, kb), jnp.maximum(ka, kb)]
            else:
                klo, xlo, khi, xhi = _cmpex(ka, x[b:b + d], kb,
                                            x[b + d:b + 2 * d])
                kparts += [klo, khi]
                xparts += [xlo, xhi]
        k = jnp.concatenate(kparts, axis=0)
        if x is not None:
            x = jnp.concatenate(xparts, axis=0)
        d //= 2
    return k, x


def _repack(h, pw, side):
    if pw == 0:
        if side == 0:
            return h
        hi = jax.lax.bitcast_convert_type(h, jnp.int32) + 1
        return jax.lax.bitcast_convert_type(hi, jnp.float32)
    low = (1 << pw) - 1
    hi = jax.lax.bitcast_convert_type(h, jnp.int32)
    hi = hi + (hi & low) + side
    return jax.lax.bitcast_convert_type(hi, jnp.float32)


def _merge_packed(a, pw, truncate):
    R, L, _ = a.shape
    A = _repack(a[:, :L // 2], pw, 0)
    Br = _rev0(_repack(a[:, L // 2:], pw, 1))
    if truncate:
        lo, _ = _bitonic_clean(jnp.minimum(A, Br), None)
        return lo
    v = jnp.concatenate([A, Br], axis=0)
    v, _ = _bitonic_clean(v, None)
    return v


def _merge_kv(k, x, truncate):
    R, L, _ = k.shape
    ka, xa = k[:, :L // 2], x[:, :L // 2]
    kb = _rev0(k[:, L // 2:])
    xb = _rev0(x[:, L // 2:])
    if truncate:
        klo, xlo, _, _ = _cmpex(ka, xa, kb, xb)
        return _bitonic_clean(klo, xlo)
    kc = jnp.concatenate([ka, kb], axis=0)
    xc = jnp.concatenate([xa, xb], axis=0)
    return _bitonic_clean(kc, xc)


_UNPACK_L = 16


def _knn_body(pfull_ref, ptile_ref, idx_ref):
    p_all = pfull_ref[...]
    p_til = ptile_ref[...]
    T = p_til.shape[0]
    sq = jnp.sum(p_all * p_all, axis=1)
    sqt = jnp.sum(p_til * p_til, axis=1)
    dot = jax.lax.dot_general(p_all, p_til, (((1,), (1,)), ((), ())),
                              preferred_element_type=jnp.float32)
    key = (sq[:, None] - 2.0 * dot) + sqt[None, :]
    key = jnp.maximum(key, 1e-30)
    kb = jax.lax.bitcast_convert_type(key, jnp.int32)
    kb = kb & jnp.int32(~0xFF)
    a = jax.lax.bitcast_convert_type(kb, jnp.float32)

    a = a.reshape(1, N, T)
    pw = 0
    while a.shape[0] < K:
        a = _merge_packed(a, pw, truncate=False)
        pw += 1
    while a.shape[1] > _UNPACK_L:
        a = _merge_packed(a, pw, truncate=True)
        pw += 1
    L = a.shape[1]
    ai = jax.lax.bitcast_convert_type(a, jnp.int32)
    m = ai & jnp.int32((1 << pw) - 1)
    l_iota = jax.lax.broadcasted_iota(jnp.int32, a.shape, 1)
    x = m * L + l_iota
    k = a
    while k.shape[1] > 1:
        k, x = _merge_kv(k, x, truncate=True)
    idx_ref[...] = x[:, 0, :]


def _knn(p_b):
    T = 128
    grid = (N // T,)
    return pl.pallas_call(
        _knn_body,
        grid=grid,
        in_specs=[
            pl.BlockSpec((N, 3), lambda i: (0, 0)),
            pl.BlockSpec((T, 3), lambda i: (i, 0)),
        ],
        out_specs=pl.BlockSpec((K, T), lambda i: (0, i)),
        out_shape=jax.ShapeDtypeStruct((K, N), jnp.int32),
    )(p_b, p_b)


_N_IDX = K * N
_SC_W = 32
_PER_W = _N_IDX // _SC_W
_CHUNK = 512
_ROW_W = 2 * D_HID


def _sc_gather_body(t_ref, idx_ref, out_ref, idx_v, rows_v, sem):
    wid = jax.lax.axis_index("c") * 16 + jax.lax.axis_index("s")
    base = wid * _PER_W

    def body(i, carry):
        off = base + i * _CHUNK
        pltpu.sync_copy(idx_ref.at[pl.ds(off, _CHUNK)], idx_v)
        pltpu.async_copy(t_ref.at[idx_v], rows_v, sem).wait()
        pltpu.sync_copy(rows_v, out_ref.at[pl.ds(off, _CHUNK)])
        return carry

    jax.lax.fori_loop(0, _PER_W // _CHUNK, body, 0)


def _sc_gather(t_flat, idx_flat):
    mesh = plsc.VectorSubcoreMesh(core_axis_name="c", subcore_axis_name="s")
    fn = functools.partial(
        pl.kernel,
        mesh=mesh,
        out_type=jax.ShapeDtypeStruct((_N_IDX, _ROW_W), jnp.float32),
        scratch_types=[
            pltpu.VMEM((_CHUNK,), jnp.int32),
            pltpu.VMEM((_CHUNK, _ROW_W), jnp.float32),
            pltpu.SemaphoreType.DMA,
        ],
    )(_sc_gather_body)
    return fn(t_flat, idx_flat)


def kernel(f, p, W_sc, b_sc, W_pre, b_pre, W_p2f, b_p2f, W_mlp, b_mlp,
           W_pst, b_pst):
    A = W_p2f[0:3] - W_p2f[6:9]
    Bm = W_p2f[3:6] + W_p2f[6:9]
    W_pab = jnp.concatenate([A, Bm], axis=1)

    tc_pack, res = _prologue(f, p, W_pre, b_pre, W_sc, b_sc, W_pab, b_p2f)

    outs = []
    for b in range(B):
        idx_b = _knn(p[b])
        tg_b = _sc_gather(tc_pack[b], idx_b.reshape(-1))
        tg_b = tg_b.reshape(K, N, _ROW_W)
        outs.append(_tail(tg_b, tc_pack[b], res[b], W_mlp, b_mlp,
                          W_pst, b_pst))
    return (jnp.stack(outs), p)


# speedup vs baseline: 1.4942x; 1.0206x over previous
import functools
import jax
import jax.numpy as jnp
from jax.experimental import pallas as pl
from jax.experimental.pallas import tpu as pltpu
from jax.experimental.pallas import tpu_sc as plsc

B, N, D_IN, D_OUT, D_HID = 4, 4096, 128, 256, 64
K = 32


def _prologue_body(f_ref, p_ref, wpre_ref, bpre_ref, wsc_ref, bsc_ref,
                   wpab_ref, bp2f_ref, tc_ref, res_ref):
    f = f_ref[0]
    p = p_ref[0]
    h = f @ wpre_ref[...] + bpre_ref[...]
    h = jnp.maximum(h, 0.01 * h)
    pa = p @ wpab_ref[...]
    t = h + pa[:, D_HID:]
    c = pa[:, :D_HID] - h + bp2f_ref[...]
    tc_ref[0] = jnp.concatenate([t, c], axis=1)
    res_ref[0] = f @ wsc_ref[...] + bsc_ref[...]


def _prologue(f, p, W_pre, b_pre, W_sc, b_sc, W_pab, b_p2f):
    T = 1024
    grid = (B, N // T)
    return pl.pallas_call(
        _prologue_body,
        grid=grid,
        in_specs=[
            pl.BlockSpec((1, T, D_IN), lambda b, i: (b, i, 0)),
            pl.BlockSpec((1, T, 3), lambda b, i: (b, i, 0)),
            pl.BlockSpec((D_IN, D_HID), lambda b, i: (0, 0)),
            pl.BlockSpec((D_HID,), lambda b, i: (0,)),
            pl.BlockSpec((D_IN, D_OUT), lambda b, i: (0, 0)),
            pl.BlockSpec((D_OUT,), lambda b, i: (0,)),
            pl.BlockSpec((3, 2 * D_HID), lambda b, i: (0, 0)),
            pl.BlockSpec((D_HID,), lambda b, i: (0,)),
        ],
        out_specs=[
            pl.BlockSpec((1, T, 2 * D_HID), lambda b, i: (b, i, 0)),
            pl.BlockSpec((1, T, D_OUT), lambda b, i: (b, i, 0)),
        ],
        out_shape=[
            jax.ShapeDtypeStruct((B, N, 2 * D_HID), jnp.float32),
            jax.ShapeDtypeStruct((B, N, D_OUT), jnp.float32),
        ],
    )(f, p, W_pre, b_pre, W_sc, b_sc, W_pab, b_p2f)


def _tail_body(tg_ref, c_ref, res_ref, wmlp_ref, bmlp_ref, wpst_ref,
               bpst_ref, out_ref):
    Trows = c_ref.shape[0]
    tg = tg_ref[:, :, :D_HID]
    c = c_ref[:, D_HID:]
    g = tg + c[None, :, :]
    g = jnp.maximum(g, 0.01 * g)
    z = g.reshape(K * Trows, D_HID) @ wmlp_ref[...] + bmlp_ref[...]
    z = jnp.maximum(z, 0.01 * z)
    m = jnp.max(z.reshape(K, Trows, D_HID), axis=0)
    o = m @ wpst_ref[...] + bpst_ref[...] + res_ref[...]
    out_ref[...] = jnp.maximum(o, 0.01 * o)


def _tail(tg_b, tc_b, res_b, W_mlp, b_mlp, W_pst, b_pst):
    T = 512
    grid = (N // T,)
    return pl.pallas_call(
        _tail_body,
        grid=grid,
        in_specs=[
            pl.BlockSpec((K, T, 2 * D_HID), lambda i: (0, i, 0)),
            pl.BlockSpec((T, 2 * D_HID), lambda i: (i, 0)),
            pl.BlockSpec((T, D_OUT), lambda i: (i, 0)),
            pl.BlockSpec((D_HID, D_HID), lambda i: (0, 0)),
            pl.BlockSpec((D_HID,), lambda i: (0,)),
            pl.BlockSpec((D_HID, D_OUT), lambda i: (0, 0)),
            pl.BlockSpec((D_OUT,), lambda i: (0,)),
        ],
        out_specs=pl.BlockSpec((T, D_OUT), lambda i: (i, 0)),
        out_shape=jax.ShapeDtypeStruct((N, D_OUT), jnp.float32),
    )(tg_b, tc_b, res_b, W_mlp, b_mlp, W_pst, b_pst)


def _rev0(a):
    if a.shape[0] == 1:
        return a
    return jnp.concatenate([a[i:i + 1] for i in range(a.shape[0] - 1, -1, -1)],
                           axis=0)


def _cmpex(ka, xa, kb, xb):
    m = kb < ka
    klo = jnp.where(m, kb, ka)
    khi = jnp.where(m, ka, kb)
    xlo = jnp.where(m, xb, xa)
    xhi = jnp.where(m, xa, xb)
    return klo, xlo, khi, xhi


def _bitonic_clean(k, x):
    R = k.shape[0]
    d = R // 2
    while d >= 1:
        kparts, xparts = [], []
        for b in range(0, R, 2 * d):
            ka, kb = k[b:b + d], k[b + d:b + 2 * d]
            if x is None:
                kparts += [jnp.minimum(ka, kb), jnp.maximum(ka, kb)]
            else:
                klo, xlo, khi, xhi = _cmpex(ka, x[b:b + d], kb,
                                            x[b + d:b + 2 * d])
                kparts += [klo, khi]
                xparts += [xlo, xhi]
        k = jnp.concatenate(kparts, axis=0)
        if x is not None:
            x = jnp.concatenate(xparts, axis=0)
        d //= 2
    return k, x


def _repack(h, pw, side):
    if pw == 0:
        if side == 0:
            return h
        hi = jax.lax.bitcast_convert_type(h, jnp.int32) + 1
        return jax.lax.bitcast_convert_type(hi, jnp.float32)
    low = (1 << pw) - 1
    hi = jax.lax.bitcast_convert_type(h, jnp.int32)
    hi = hi + (hi & low) + side
    return jax.lax.bitcast_convert_type(hi, jnp.float32)


def _merge_packed(a, pw, truncate):
    R, L, _ = a.shape
    A = _repack(a[:, :L // 2], pw, 0)
    Br = _rev0(_repack(a[:, L // 2:], pw, 1))
    if truncate:
        lo, _ = _bitonic_clean(jnp.minimum(A, Br), None)
        return lo
    v = jnp.concatenate([A, Br], axis=0)
    v, _ = _bitonic_clean(v, None)
    return v


def _merge_kv(k, x, truncate):
    R, L, _ = k.shape
    ka, xa = k[:, :L // 2], x[:, :L // 2]
    kb = _rev0(k[:, L // 2:])
    xb = _rev0(x[:, L // 2:])
    if truncate:
        klo, xlo, _, _ = _cmpex(ka, xa, kb, xb)
        return _bitonic_clean(klo, xlo)
    kc = jnp.concatenate([ka, kb], axis=0)
    xc = jnp.concatenate([xa, xb], axis=0)
    return _bitonic_clean(kc, xc)


_UNPACK_L = 16


def _bitonic_sort_r(a):
    R = a.shape[0]
    k = 2
    while k <= R:
        d = k // 2
        while d >= 1:
            parts = []
            for base in range(0, R, 2 * d):
                lo = a[base:base + d]
                hi = a[base + d:base + 2 * d]
                if (base & k) == 0:
                    parts += [jnp.minimum(lo, hi), jnp.maximum(lo, hi)]
                else:
                    parts += [jnp.maximum(lo, hi), jnp.minimum(lo, hi)]
            a = jnp.concatenate(parts, axis=0)
            d //= 2
        k *= 2
    return a


def _knn_body(pfull_ref, ptile_ref, idx_ref):
    p_all = pfull_ref[...]
    p_til = ptile_ref[...]
    T = p_til.shape[0]
    sq = jnp.sum(p_all * p_all, axis=1)
    sqt = jnp.sum(p_til * p_til, axis=1)
    dot = jax.lax.dot_general(p_all, p_til, (((1,), (1,)), ((), ())),
                              preferred_element_type=jnp.float32)
    key = (sq[:, None] - 2.0 * dot) + sqt[None, :]
    key = jnp.maximum(key, 1e-30)
    kb = jax.lax.bitcast_convert_type(key, jnp.int32)
    kb = kb & jnp.int32(~0xFF)
    kb = kb.reshape(K, N // K, T)
    kb = kb + jax.lax.broadcasted_iota(jnp.int32, kb.shape, 0)
    a = jax.lax.bitcast_convert_type(kb, jnp.float32)

    a = _bitonic_sort_r(a)
    pw = 5
    while a.shape[1] > _UNPACK_L:
        a = _merge_packed(a, pw, truncate=True)
        pw += 1
    L = a.shape[1]
    ai = jax.lax.bitcast_convert_type(a, jnp.int32)
    m = ai & jnp.int32((1 << pw) - 1)
    l_iota = jax.lax.broadcasted_iota(jnp.int32, a.shape, 1)
    x = m * L + l_iota
    k = a
    while k.shape[1] > 1:
        k, x = _merge_kv(k, x, truncate=True)
    idx_ref[...] = x[:, 0, :]


def _knn(p_b):
    T = 128
    grid = (N // T,)
    return pl.pallas_call(
        _knn_body,
        grid=grid,
        in_specs=[
            pl.BlockSpec((N, 3), lambda i: (0, 0)),
            pl.BlockSpec((T, 3), lambda i: (i, 0)),
        ],
        out_specs=pl.BlockSpec((K, T), lambda i: (0, i)),
        out_shape=jax.ShapeDtypeStruct((K, N), jnp.int32),
    )(p_b, p_b)


_N_IDX = K * N
_SC_W = 32
_PER_W = _N_IDX // _SC_W
_CHUNK = 256
_ROW_W = 2 * D_HID


def _sc_gather_body(t_ref, idx_ref, out_ref, idx0, idx1, rows0, rows1,
                    gsem0, gsem1, wsem0, wsem1):
    wid = jax.lax.axis_index("c") * 16 + jax.lax.axis_index("s")
    base = wid * _PER_W
    idx_v = (idx0, idx1)
    rows_v = (rows0, rows1)
    gsem = (gsem0, gsem1)
    wsem = (wsem0, wsem1)
    n_it = _PER_W // _CHUNK
    writes = [None] * n_it
    for i in range(n_it):
        bi = i & 1
        if i >= 2:
            writes[i - 2].wait()
        off = base + i * _CHUNK
        pltpu.sync_copy(idx_ref.at[pl.ds(off, _CHUNK)], idx_v[bi])
        pltpu.async_copy(t_ref.at[idx_v[bi]], rows_v[bi], gsem[bi]).wait()
        writes[i] = pltpu.async_copy(rows_v[bi], out_ref.at[pl.ds(off, _CHUNK)],
                                     wsem[bi])
    writes[n_it - 2].wait()
    writes[n_it - 1].wait()


def _sc_gather(t_flat, idx_flat):
    mesh = plsc.VectorSubcoreMesh(core_axis_name="c", subcore_axis_name="s")
    fn = functools.partial(
        pl.kernel,
        mesh=mesh,
        out_type=jax.ShapeDtypeStruct((_N_IDX, _ROW_W), jnp.float32),
        scratch_types=[
            pltpu.VMEM((_CHUNK,), jnp.int32),
            pltpu.VMEM((_CHUNK,), jnp.int32),
            pltpu.VMEM((_CHUNK, _ROW_W), jnp.float32),
            pltpu.VMEM((_CHUNK, _ROW_W), jnp.float32),
            pltpu.SemaphoreType.DMA,
            pltpu.SemaphoreType.DMA,
            pltpu.SemaphoreType.DMA,
            pltpu.SemaphoreType.DMA,
        ],
    )(_sc_gather_body)
    return fn(t_flat, idx_flat)


def kernel(f, p, W_sc, b_sc, W_pre, b_pre, W_p2f, b_p2f, W_mlp, b_mlp,
           W_pst, b_pst):
    A = W_p2f[0:3] - W_p2f[6:9]
    Bm = W_p2f[3:6] + W_p2f[6:9]
    W_pab = jnp.concatenate([A, Bm], axis=1)

    tc_pack, res = _prologue(f, p, W_pre, b_pre, W_sc, b_sc, W_pab, b_p2f)

    outs = []
    for b in range(B):
        idx_b = _knn(p[b])
        tg_b = _sc_gather(tc_pack[b], idx_b.reshape(-1))
        tg_b = tg_b.reshape(K, N, _ROW_W)
        outs.append(_tail(tg_b, tc_pack[b], res[b], W_mlp, b_mlp,
                          W_pst, b_pst))
    return (jnp.stack(outs), p)


# speedup vs baseline: 1.6118x; 1.0787x over previous
import functools
import jax
import jax.numpy as jnp
from jax.experimental import pallas as pl
from jax.experimental.pallas import tpu as pltpu
from jax.experimental.pallas import tpu_sc as plsc

B, N, D_IN, D_OUT, D_HID = 4, 4096, 128, 256, 64
K = 32


def _prologue_body(f_ref, p_ref, wpre_ref, bpre_ref, wsc_ref, bsc_ref,
                   wpab_ref, bp2f_ref, tc_ref, res_ref):
    f = f_ref[0]
    p = p_ref[0]
    h = f @ wpre_ref[...] + bpre_ref[...]
    h = jnp.maximum(h, 0.01 * h)
    pa = p @ wpab_ref[...]
    t = h + pa[:, D_HID:]
    c = pa[:, :D_HID] - h + bp2f_ref[...]
    tc_ref[0] = jnp.concatenate([t, c], axis=1)
    res_ref[0] = f @ wsc_ref[...] + bsc_ref[...]


def _prologue(f, p, W_pre, b_pre, W_sc, b_sc, W_pab, b_p2f):
    T = 1024
    grid = (B, N // T)
    return pl.pallas_call(
        _prologue_body,
        grid=grid,
        in_specs=[
            pl.BlockSpec((1, T, D_IN), lambda b, i: (b, i, 0)),
            pl.BlockSpec((1, T, 3), lambda b, i: (b, i, 0)),
            pl.BlockSpec((D_IN, D_HID), lambda b, i: (0, 0)),
            pl.BlockSpec((D_HID,), lambda b, i: (0,)),
            pl.BlockSpec((D_IN, D_OUT), lambda b, i: (0, 0)),
            pl.BlockSpec((D_OUT,), lambda b, i: (0,)),
            pl.BlockSpec((3, 2 * D_HID), lambda b, i: (0, 0)),
            pl.BlockSpec((D_HID,), lambda b, i: (0,)),
        ],
        out_specs=[
            pl.BlockSpec((1, T, 2 * D_HID), lambda b, i: (b, i, 0)),
            pl.BlockSpec((1, T, D_OUT), lambda b, i: (b, i, 0)),
        ],
        out_shape=[
            jax.ShapeDtypeStruct((B, N, 2 * D_HID), jnp.float32),
            jax.ShapeDtypeStruct((B, N, D_OUT), jnp.float32),
        ],
    )(f, p, W_pre, b_pre, W_sc, b_sc, W_pab, b_p2f)


def _tail_body(tg_ref, c_ref, res_ref, wmlp_ref, bmlp_ref, wpst_ref,
               bpst_ref, out_ref):
    Trows = c_ref.shape[0]
    tg = tg_ref[:, :, :D_HID]
    c = c_ref[:, D_HID:]
    g = tg + c[None, :, :]
    g = jnp.maximum(g, 0.01 * g)
    z = g.reshape(K * Trows, D_HID) @ wmlp_ref[...] + bmlp_ref[...]
    z = jnp.maximum(z, 0.01 * z)
    m = jnp.max(z.reshape(K, Trows, D_HID), axis=0)
    o = m @ wpst_ref[...] + bpst_ref[...] + res_ref[...]
    out_ref[...] = jnp.maximum(o, 0.01 * o)


def _tail(tg_b, tc_b, res_b, W_mlp, b_mlp, W_pst, b_pst):
    T = 512
    grid = (N // T,)
    return pl.pallas_call(
        _tail_body,
        grid=grid,
        in_specs=[
            pl.BlockSpec((K, T, 2 * D_HID), lambda i: (0, i, 0)),
            pl.BlockSpec((T, 2 * D_HID), lambda i: (i, 0)),
            pl.BlockSpec((T, D_OUT), lambda i: (i, 0)),
            pl.BlockSpec((D_HID, D_HID), lambda i: (0, 0)),
            pl.BlockSpec((D_HID,), lambda i: (0,)),
            pl.BlockSpec((D_HID, D_OUT), lambda i: (0, 0)),
            pl.BlockSpec((D_OUT,), lambda i: (0,)),
        ],
        out_specs=pl.BlockSpec((T, D_OUT), lambda i: (i, 0)),
        out_shape=jax.ShapeDtypeStruct((N, D_OUT), jnp.float32),
    )(tg_b, tc_b, res_b, W_mlp, b_mlp, W_pst, b_pst)


def _rev0(a):
    if a.shape[0] == 1:
        return a
    return jnp.concatenate([a[i:i + 1] for i in range(a.shape[0] - 1, -1, -1)],
                           axis=0)


def _cmpex(ka, xa, kb, xb):
    m = kb < ka
    klo = jnp.where(m, kb, ka)
    khi = jnp.where(m, ka, kb)
    xlo = jnp.where(m, xb, xa)
    xhi = jnp.where(m, xa, xb)
    return klo, xlo, khi, xhi


def _bitonic_clean(k, x):
    R = k.shape[0]
    d = R // 2
    while d >= 1:
        kparts, xparts = [], []
        for b in range(0, R, 2 * d):
            ka, kb = k[b:b + d], k[b + d:b + 2 * d]
            if x is None:
                kparts += [jnp.minimum(ka, kb), jnp.maximum(ka, kb)]
            else:
                klo, xlo, khi, xhi = _cmpex(ka, x[b:b + d], kb,
                                            x[b + d:b + 2 * d])
                kparts += [klo, khi]
                xparts += [xlo, xhi]
        k = jnp.concatenate(kparts, axis=0)
        if x is not None:
            x = jnp.concatenate(xparts, axis=0)
        d //= 2
    return k, x


def _repack(h, pw, side):
    if pw == 0:
        if side == 0:
            return h
        hi = jax.lax.bitcast_convert_type(h, jnp.int32) + 1
        return jax.lax.bitcast_convert_type(hi, jnp.float32)
    low = (1 << pw) - 1
    hi = jax.lax.bitcast_convert_type(h, jnp.int32)
    hi = hi + (hi & low) + side
    return jax.lax.bitcast_convert_type(hi, jnp.float32)


def _merge_packed(a, pw, truncate):
    R, L, _ = a.shape
    A = _repack(a[:, :L // 2], pw, 0)
    Br = _rev0(_repack(a[:, L // 2:], pw, 1))
    if truncate:
        lo, _ = _bitonic_clean(jnp.minimum(A, Br), None)
        return lo
    v = jnp.concatenate([A, Br], axis=0)
    v, _ = _bitonic_clean(v, None)
    return v


def _merge_kv(k, x, truncate):
    R, L, _ = k.shape
    ka, xa = k[:, :L // 2], x[:, :L // 2]
    kb = _rev0(k[:, L // 2:])
    xb = _rev0(x[:, L // 2:])
    if truncate:
        klo, xlo, _, _ = _cmpex(ka, xa, kb, xb)
        return _bitonic_clean(klo, xlo)
    kc = jnp.concatenate([ka, kb], axis=0)
    xc = jnp.concatenate([xa, xb], axis=0)
    return _bitonic_clean(kc, xc)


_UNPACK_L = 16


def _batcher_pairs(n):
    pairs = []

    def merge(lo, m, r):
        step = r * 2
        if step < m:
            merge(lo, m, step)
            merge(lo + r, m, step)
            for i in range(lo + r, lo + m - r, step):
                pairs.append((i, i + r))
        else:
            pairs.append((lo, lo + r))

    def sort(lo, m):
        if m > 1:
            half = m // 2
            sort(lo, half)
            sort(lo + half, half)
            merge(lo, m, 1)

    sort(0, n)
    return pairs


def _sort_net_list(vals):
    for i, j in _batcher_pairs(len(vals)):
        mn = jnp.minimum(vals[i], vals[j])
        mx = jnp.maximum(vals[i], vals[j])
        vals[i], vals[j] = mn, mx
    return vals


def _bitonic_sort_r(a):
    R, L, T = a.shape
    CH = 8
    out_chunks = []
    for l0 in range(0, L, CH):
        vals = [a[r, l0:l0 + CH] for r in range(R)]
        vals = _sort_net_list(vals)
        out_chunks.append(jnp.concatenate([v[None] for v in vals], axis=0))
    return jnp.concatenate(out_chunks, axis=1)


def _knn_body(pfull_ref, ptile_ref, idx_ref):
    p_all = pfull_ref[...]
    p_til = ptile_ref[...]
    T = p_til.shape[0]
    sq = jnp.sum(p_all * p_all, axis=1)
    sqt = jnp.sum(p_til * p_til, axis=1)
    dot = jax.lax.dot_general(p_all, p_til, (((1,), (1,)), ((), ())),
                              preferred_element_type=jnp.float32)
    key = (sq[:, None] - 2.0 * dot) + sqt[None, :]
    key = jnp.maximum(key, 1e-30)
    kb = jax.lax.bitcast_convert_type(key, jnp.int32)
    kb = kb & jnp.int32(~0xFF)
    kb = kb.reshape(K, N // K, T)
    kb = kb + jax.lax.broadcasted_iota(jnp.int32, kb.shape, 0)
    a = jax.lax.bitcast_convert_type(kb, jnp.float32)

    a = _bitonic_sort_r(a)
    pw = 5
    while a.shape[1] > _UNPACK_L:
        a = _merge_packed(a, pw, truncate=True)
        pw += 1
    L = a.shape[1]
    ai = jax.lax.bitcast_convert_type(a, jnp.int32)
    m = ai & jnp.int32((1 << pw) - 1)
    l_iota = jax.lax.broadcasted_iota(jnp.int32, a.shape, 1)
    x = m * L + l_iota
    k = a
    while k.shape[1] > 1:
        k, x = _merge_kv(k, x, truncate=True)
    idx_ref[...] = x[:, 0, :]


def _knn(p_b):
    T = 128
    grid = (N // T,)
    return pl.pallas_call(
        _knn_body,
        grid=grid,
        in_specs=[
            pl.BlockSpec((N, 3), lambda i: (0, 0)),
            pl.BlockSpec((T, 3), lambda i: (i, 0)),
        ],
        out_specs=pl.BlockSpec((K, T), lambda i: (0, i)),
        out_shape=jax.ShapeDtypeStruct((K, N), jnp.int32),
    )(p_b, p_b)


_N_IDX = K * N
_SC_W = 32
_PER_W = _N_IDX // _SC_W
_CHUNK = 256
_ROW_W = 2 * D_HID


def _sc_gather_body(t_ref, idx_ref, out_ref, idx0, idx1, rows0, rows1,
                    gsem0, gsem1, wsem0, wsem1):
    wid = jax.lax.axis_index("c") * 16 + jax.lax.axis_index("s")
    base = wid * _PER_W
    idx_v = (idx0, idx1)
    rows_v = (rows0, rows1)
    gsem = (gsem0, gsem1)
    wsem = (wsem0, wsem1)
    n_it = _PER_W // _CHUNK
    writes = [None] * n_it
    for i in range(n_it):
        bi = i & 1
        if i >= 2:
            writes[i - 2].wait()
        off = base + i * _CHUNK
        pltpu.sync_copy(idx_ref.at[pl.ds(off, _CHUNK)], idx_v[bi])
        pltpu.async_copy(t_ref.at[idx_v[bi]], rows_v[bi], gsem[bi]).wait()
        writes[i] = pltpu.async_copy(rows_v[bi], out_ref.at[pl.ds(off, _CHUNK)],
                                     wsem[bi])
    writes[n_it - 2].wait()
    writes[n_it - 1].wait()


def _sc_gather(t_flat, idx_flat):
    mesh = plsc.VectorSubcoreMesh(core_axis_name="c", subcore_axis_name="s")
    fn = functools.partial(
        pl.kernel,
        mesh=mesh,
        out_type=jax.ShapeDtypeStruct((_N_IDX, _ROW_W), jnp.float32),
        scratch_types=[
            pltpu.VMEM((_CHUNK,), jnp.int32),
            pltpu.VMEM((_CHUNK,), jnp.int32),
            pltpu.VMEM((_CHUNK, _ROW_W), jnp.float32),
            pltpu.VMEM((_CHUNK, _ROW_W), jnp.float32),
            pltpu.SemaphoreType.DMA,
            pltpu.SemaphoreType.DMA,
            pltpu.SemaphoreType.DMA,
            pltpu.SemaphoreType.DMA,
        ],
    )(_sc_gather_body)
    return fn(t_flat, idx_flat)


def kernel(f, p, W_sc, b_sc, W_pre, b_pre, W_p2f, b_p2f, W_mlp, b_mlp,
           W_pst, b_pst):
    A = W_p2f[0:3] - W_p2f[6:9]
    Bm = W_p2f[3:6] + W_p2f[6:9]
    W_pab = jnp.concatenate([A, Bm], axis=1)

    tc_pack, res = _prologue(f, p, W_pre, b_pre, W_sc, b_sc, W_pab, b_p2f)

    outs = []
    for b in range(B):
        idx_b = _knn(p[b])
        tg_b = _sc_gather(tc_pack[b], idx_b.reshape(-1))
        tg_b = tg_b.reshape(K, N, _ROW_W)
        outs.append(_tail(tg_b, tc_pack[b], res[b], W_mlp, b_mlp,
                          W_pst, b_pst))
    return (jnp.stack(outs), p)


# speedup vs baseline: 1.6128x; 1.0006x over previous
import functools
import jax
import jax.numpy as jnp
from jax.experimental import pallas as pl
from jax.experimental.pallas import tpu as pltpu
from jax.experimental.pallas import tpu_sc as plsc

B, N, D_IN, D_OUT, D_HID = 4, 4096, 128, 256, 64
K = 32


def _prologue_body(f_ref, p_ref, wpre_ref, bpre_ref, wsc_ref, bsc_ref,
                   wpab_ref, bp2f_ref, tc_ref, res_ref):
    f = f_ref[0]
    p = p_ref[0]
    h = f @ wpre_ref[...] + bpre_ref[...]
    h = jnp.maximum(h, 0.01 * h)
    pa = p @ wpab_ref[...]
    t = h + pa[:, D_HID:]
    c = pa[:, :D_HID] - h + bp2f_ref[...]
    tc_ref[0] = jnp.concatenate([t, c], axis=1)
    res_ref[0] = f @ wsc_ref[...] + bsc_ref[...]


def _prologue(f, p, W_pre, b_pre, W_sc, b_sc, W_pab, b_p2f):
    T = 1024
    grid = (B, N // T)
    return pl.pallas_call(
        _prologue_body,
        grid=grid,
        in_specs=[
            pl.BlockSpec((1, T, D_IN), lambda b, i: (b, i, 0)),
            pl.BlockSpec((1, T, 3), lambda b, i: (b, i, 0)),
            pl.BlockSpec((D_IN, D_HID), lambda b, i: (0, 0)),
            pl.BlockSpec((D_HID,), lambda b, i: (0,)),
            pl.BlockSpec((D_IN, D_OUT), lambda b, i: (0, 0)),
            pl.BlockSpec((D_OUT,), lambda b, i: (0,)),
            pl.BlockSpec((3, 2 * D_HID), lambda b, i: (0, 0)),
            pl.BlockSpec((D_HID,), lambda b, i: (0,)),
        ],
        out_specs=[
            pl.BlockSpec((1, T, 2 * D_HID), lambda b, i: (b, i, 0)),
            pl.BlockSpec((1, T, D_OUT), lambda b, i: (b, i, 0)),
        ],
        out_shape=[
            jax.ShapeDtypeStruct((B, N, 2 * D_HID), jnp.float32),
            jax.ShapeDtypeStruct((B, N, D_OUT), jnp.float32),
        ],
    )(f, p, W_pre, b_pre, W_sc, b_sc, W_pab, b_p2f)


def _tail_body(tg_ref, c_ref, res_ref, wmlp_ref, bmlp_ref, wpst_ref,
               bpst_ref, out_ref):
    Trows = c_ref.shape[0]
    tg = tg_ref[:, :, :D_HID]
    c = c_ref[:, D_HID:]
    g = tg + c[None, :, :]
    g = jnp.maximum(g, 0.01 * g)
    z = g.reshape(K * Trows, D_HID) @ wmlp_ref[...] + bmlp_ref[...]
    z = jnp.maximum(z, 0.01 * z)
    m = jnp.max(z.reshape(K, Trows, D_HID), axis=0)
    o = m @ wpst_ref[...] + bpst_ref[...] + res_ref[...]
    out_ref[...] = jnp.maximum(o, 0.01 * o)


def _tail(tg_b, tc_b, res_b, W_mlp, b_mlp, W_pst, b_pst):
    T = 512
    grid = (N // T,)
    return pl.pallas_call(
        _tail_body,
        grid=grid,
        in_specs=[
            pl.BlockSpec((K, T, 2 * D_HID), lambda i: (0, i, 0)),
            pl.BlockSpec((T, 2 * D_HID), lambda i: (i, 0)),
            pl.BlockSpec((T, D_OUT), lambda i: (i, 0)),
            pl.BlockSpec((D_HID, D_HID), lambda i: (0, 0)),
            pl.BlockSpec((D_HID,), lambda i: (0,)),
            pl.BlockSpec((D_HID, D_OUT), lambda i: (0, 0)),
            pl.BlockSpec((D_OUT,), lambda i: (0,)),
        ],
        out_specs=pl.BlockSpec((T, D_OUT), lambda i: (i, 0)),
        out_shape=jax.ShapeDtypeStruct((N, D_OUT), jnp.float32),
    )(tg_b, tc_b, res_b, W_mlp, b_mlp, W_pst, b_pst)


def _rev0(a):
    if a.shape[0] == 1:
        return a
    return jnp.concatenate([a[i:i + 1] for i in range(a.shape[0] - 1, -1, -1)],
                           axis=0)


def _cmpex(ka, xa, kb, xb):
    m = kb < ka
    klo = jnp.where(m, kb, ka)
    khi = jnp.where(m, ka, kb)
    xlo = jnp.where(m, xb, xa)
    xhi = jnp.where(m, xa, xb)
    return klo, xlo, khi, xhi


def _bitonic_clean(k, x):
    R = k.shape[0]
    d = R // 2
    while d >= 1:
        kparts, xparts = [], []
        for b in range(0, R, 2 * d):
            ka, kb = k[b:b + d], k[b + d:b + 2 * d]
            if x is None:
                kparts += [jnp.minimum(ka, kb), jnp.maximum(ka, kb)]
            else:
                klo, xlo, khi, xhi = _cmpex(ka, x[b:b + d], kb,
                                            x[b + d:b + 2 * d])
                kparts += [klo, khi]
                xparts += [xlo, xhi]
        k = jnp.concatenate(kparts, axis=0)
        if x is not None:
            x = jnp.concatenate(xparts, axis=0)
        d //= 2
    return k, x


def _repack(h, pw, side):
    if pw == 0:
        if side == 0:
            return h
        hi = jax.lax.bitcast_convert_type(h, jnp.int32) + 1
        return jax.lax.bitcast_convert_type(hi, jnp.float32)
    low = (1 << pw) - 1
    hi = jax.lax.bitcast_convert_type(h, jnp.int32)
    hi = hi + (hi & low) + side
    return jax.lax.bitcast_convert_type(hi, jnp.float32)


def _merge_packed(a, pw, truncate):
    R, L, _ = a.shape
    A = _repack(a[:, :L // 2], pw, 0)
    Br = _rev0(_repack(a[:, L // 2:], pw, 1))
    if truncate:
        lo, _ = _bitonic_clean(jnp.minimum(A, Br), None)
        return lo
    v = jnp.concatenate([A, Br], axis=0)
    v, _ = _bitonic_clean(v, None)
    return v


def _merge_kv(k, x, truncate):
    R, L, _ = k.shape
    ka, xa = k[:, :L // 2], x[:, :L // 2]
    kb = _rev0(k[:, L // 2:])
    xb = _rev0(x[:, L // 2:])
    if truncate:
        klo, xlo, _, _ = _cmpex(ka, xa, kb, xb)
        return _bitonic_clean(klo, xlo)
    kc = jnp.concatenate([ka, kb], axis=0)
    xc = jnp.concatenate([xa, xb], axis=0)
    return _bitonic_clean(kc, xc)


_UNPACK_L = 16


def _batcher_pairs(n):
    pairs = []

    def merge(lo, m, r):
        step = r * 2
        if step < m:
            merge(lo, m, step)
            merge(lo + r, m, step)
            for i in range(lo + r, lo + m - r, step):
                pairs.append((i, i + r))
        else:
            pairs.append((lo, lo + r))

    def sort(lo, m):
        if m > 1:
            half = m // 2
            sort(lo, half)
            sort(lo + half, half)
            merge(lo, m, 1)

    sort(0, n)
    return pairs


def _sort_net_list(vals):
    for i, j in _batcher_pairs(len(vals)):
        mn = jnp.minimum(vals[i], vals[j])
        mx = jnp.maximum(vals[i], vals[j])
        vals[i], vals[j] = mn, mx
    return vals


def _bitonic_sort_r(a):
    R, L, T = a.shape
    CH = 8
    out_chunks = []
    for l0 in range(0, L, CH):
        vals = [a[r, l0:l0 + CH] for r in range(R)]
        vals = _sort_net_list(vals)
        out_chunks.append(jnp.concatenate([v[None] for v in vals], axis=0))
    return jnp.concatenate(out_chunks, axis=1)


def _knn_body(pfull_ref, ptile_ref, idx_ref):
    p_all = pfull_ref[...]
    p_til = ptile_ref[...]
    T = p_til.shape[0]
    sq = jnp.sum(p_all * p_all, axis=1)
    sqt = jnp.sum(p_til * p_til, axis=1)
    dot = jax.lax.dot_general(p_all, p_til, (((1,), (1,)), ((), ())),
                              preferred_element_type=jnp.float32)
    key = (sq[:, None] - 2.0 * dot) + sqt[None, :]
    key = jnp.maximum(key, 1e-30)
    kb = jax.lax.bitcast_convert_type(key, jnp.int32)
    kb = kb & jnp.int32(~0xFF)
    kb = kb.reshape(K, N // K, T)
    kb = kb + jax.lax.broadcasted_iota(jnp.int32, kb.shape, 0)
    a = jax.lax.bitcast_convert_type(kb, jnp.float32)

    a = _bitonic_sort_r(a)
    pw = 5
    while a.shape[1] > _UNPACK_L:
        a = _merge_packed(a, pw, truncate=True)
        pw += 1
    L = a.shape[1]
    ai = jax.lax.bitcast_convert_type(a, jnp.int32)
    m = ai & jnp.int32((1 << pw) - 1)
    l_iota = jax.lax.broadcasted_iota(jnp.int32, a.shape, 1)
    x = m * L + l_iota
    k = a
    while k.shape[1] > 1:
        k, x = _merge_kv(k, x, truncate=True)
    idx_ref[...] = x[:, 0, :]


def _knn(p_b):
    T = 128
    grid = (N // T,)
    return pl.pallas_call(
        _knn_body,
        grid=grid,
        in_specs=[
            pl.BlockSpec((N, 3), lambda i: (0, 0)),
            pl.BlockSpec((T, 3), lambda i: (i, 0)),
        ],
        out_specs=pl.BlockSpec((K, T), lambda i: (0, i)),
        out_shape=jax.ShapeDtypeStruct((K, N), jnp.int32),
    )(p_b, p_b)


_N_IDX = K * N
_SC_W = 32
_PER_W = _N_IDX // _SC_W
_CHUNK = 256
_ROW_W = 2 * D_HID


def _sc_gather_body(t_ref, idx_ref, out_ref, idx0, idx1, rows0, rows1,
                    gsem0, gsem1, wsem0, wsem1):
    wid = jax.lax.axis_index("c") * 16 + jax.lax.axis_index("s")
    base = wid * _PER_W
    idx_v = (idx0, idx1)
    rows_v = (rows0, rows1)
    gsem = (gsem0, gsem1)
    wsem = (wsem0, wsem1)
    n_it = _PER_W // _CHUNK
    writes = [None] * n_it
    for i in range(n_it):
        bi = i & 1
        if i >= 2:
            writes[i - 2].wait()
        off = base + i * _CHUNK
        pltpu.sync_copy(idx_ref.at[pl.ds(off, _CHUNK)], idx_v[bi])
        pltpu.async_copy(t_ref.at[idx_v[bi]], rows_v[bi], gsem[bi]).wait()
        writes[i] = pltpu.async_copy(rows_v[bi], out_ref.at[pl.ds(off, _CHUNK)],
                                     wsem[bi])
    writes[n_it - 2].wait()
    writes[n_it - 1].wait()


def _sc_gather(t_flat, idx_flat):
    mesh = plsc.VectorSubcoreMesh(core_axis_name="c", subcore_axis_name="s")
    fn = functools.partial(
        pl.kernel,
        mesh=mesh,
        out_type=jax.ShapeDtypeStruct((_N_IDX, _ROW_W), jnp.float32),
        scratch_types=[
            pltpu.VMEM((_CHUNK,), jnp.int32),
            pltpu.VMEM((_CHUNK,), jnp.int32),
            pltpu.VMEM((_CHUNK, _ROW_W), jnp.float32),
            pltpu.VMEM((_CHUNK, _ROW_W), jnp.float32),
            pltpu.SemaphoreType.DMA,
            pltpu.SemaphoreType.DMA,
            pltpu.SemaphoreType.DMA,
            pltpu.SemaphoreType.DMA,
        ],
    )(_sc_gather_body)
    return fn(t_flat, idx_flat)


def kernel(f, p, W_sc, b_sc, W_pre, b_pre, W_p2f, b_p2f, W_mlp, b_mlp,
           W_pst, b_pst):
    A = W_p2f[0:3] - W_p2f[6:9]
    Bm = W_p2f[3:6] + W_p2f[6:9]
    W_pab = jnp.concatenate([A, Bm], axis=1)

    tc_pack, res = _prologue(f, p, W_pre, b_pre, W_sc, b_sc, W_pab, b_p2f)

    idxs = [_knn(p[b]) for b in range(B)]
    tgs = [_sc_gather(tc_pack[b], idxs[b].reshape(-1)) for b in range(B)]
    outs = [_tail(tgs[b].reshape(K, N, _ROW_W), tc_pack[b], res[b],
                  W_mlp, b_mlp, W_pst, b_pst) for b in range(B)]
    return (jnp.stack(outs), p)


# speedup vs baseline: 1.7235x; 1.0686x over previous
import functools
import jax
import jax.numpy as jnp
from jax.experimental import pallas as pl
from jax.experimental.pallas import tpu as pltpu
from jax.experimental.pallas import tpu_sc as plsc

B, N, D_IN, D_OUT, D_HID = 4, 4096, 128, 256, 64
K = 32


def _prologue_body(f_ref, p_ref, wpre_ref, bpre_ref, wsc_ref, bsc_ref,
                   wpab_ref, bp2f_ref, tc_ref, res_ref):
    f = f_ref[0]
    p = p_ref[0]
    h = f @ wpre_ref[...] + bpre_ref[...]
    h = jnp.maximum(h, 0.01 * h)
    pa = p @ wpab_ref[...]
    t = h + pa[:, D_HID:]
    c = pa[:, :D_HID] - h + bp2f_ref[...]
    tc_ref[0] = jnp.concatenate([t, c], axis=1)
    res_ref[0] = f @ wsc_ref[...] + bsc_ref[...]


def _prologue(f, p, W_pre, b_pre, W_sc, b_sc, W_pab, b_p2f):
    T = 1024
    grid = (B, N // T)
    return pl.pallas_call(
        _prologue_body,
        grid=grid,
        in_specs=[
            pl.BlockSpec((1, T, D_IN), lambda b, i: (b, i, 0)),
            pl.BlockSpec((1, T, 3), lambda b, i: (b, i, 0)),
            pl.BlockSpec((D_IN, D_HID), lambda b, i: (0, 0)),
            pl.BlockSpec((D_HID,), lambda b, i: (0,)),
            pl.BlockSpec((D_IN, D_OUT), lambda b, i: (0, 0)),
            pl.BlockSpec((D_OUT,), lambda b, i: (0,)),
            pl.BlockSpec((3, 2 * D_HID), lambda b, i: (0, 0)),
            pl.BlockSpec((D_HID,), lambda b, i: (0,)),
        ],
        out_specs=[
            pl.BlockSpec((1, T, 2 * D_HID), lambda b, i: (b, i, 0)),
            pl.BlockSpec((1, T, D_OUT), lambda b, i: (b, i, 0)),
        ],
        out_shape=[
            jax.ShapeDtypeStruct((B, N, 2 * D_HID), jnp.float32),
            jax.ShapeDtypeStruct((B, N, D_OUT), jnp.float32),
        ],
    )(f, p, W_pre, b_pre, W_sc, b_sc, W_pab, b_p2f)


def _tail_body(tg_ref, c_ref, res_ref, wmlp_ref, bmlp_ref, wpst_ref,
               bpst_ref, out_ref):
    Trows = c_ref.shape[0]
    tg = tg_ref[:, :, :D_HID]
    c = c_ref[:, D_HID:]
    g = tg + c[None, :, :]
    g = jnp.maximum(g, 0.01 * g)
    z = g.reshape(K * Trows, D_HID) @ wmlp_ref[...] + bmlp_ref[...]
    z = jnp.maximum(z, 0.01 * z)
    m = jnp.max(z.reshape(K, Trows, D_HID), axis=0)
    o = m @ wpst_ref[...] + bpst_ref[...] + res_ref[...]
    out_ref[...] = jnp.maximum(o, 0.01 * o)


def _tail(tg_b, tc_b, res_b, W_mlp, b_mlp, W_pst, b_pst):
    T = 512
    grid = (N // T,)
    return pl.pallas_call(
        _tail_body,
        grid=grid,
        in_specs=[
            pl.BlockSpec((K, T, 2 * D_HID), lambda i: (0, i, 0)),
            pl.BlockSpec((T, 2 * D_HID), lambda i: (i, 0)),
            pl.BlockSpec((T, D_OUT), lambda i: (i, 0)),
            pl.BlockSpec((D_HID, D_HID), lambda i: (0, 0)),
            pl.BlockSpec((D_HID,), lambda i: (0,)),
            pl.BlockSpec((D_HID, D_OUT), lambda i: (0, 0)),
            pl.BlockSpec((D_OUT,), lambda i: (0,)),
        ],
        out_specs=pl.BlockSpec((T, D_OUT), lambda i: (i, 0)),
        out_shape=jax.ShapeDtypeStruct((N, D_OUT), jnp.float32),
    )(tg_b, tc_b, res_b, W_mlp, b_mlp, W_pst, b_pst)


def _rev0(a):
    if a.shape[0] == 1:
        return a
    return jnp.concatenate([a[i:i + 1] for i in range(a.shape[0] - 1, -1, -1)],
                           axis=0)


def _cmpex(ka, xa, kb, xb):
    m = kb < ka
    klo = jnp.where(m, kb, ka)
    khi = jnp.where(m, ka, kb)
    xlo = jnp.where(m, xb, xa)
    xhi = jnp.where(m, xa, xb)
    return klo, xlo, khi, xhi


def _bitonic_clean(k, x):
    R = k.shape[0]
    d = R // 2
    while d >= 1:
        kparts, xparts = [], []
        for b in range(0, R, 2 * d):
            ka, kb = k[b:b + d], k[b + d:b + 2 * d]
            if x is None:
                kparts += [jnp.minimum(ka, kb), jnp.maximum(ka, kb)]
            else:
                klo, xlo, khi, xhi = _cmpex(ka, x[b:b + d], kb,
                                            x[b + d:b + 2 * d])
                kparts += [klo, khi]
                xparts += [xlo, xhi]
        k = jnp.concatenate(kparts, axis=0)
        if x is not None:
            x = jnp.concatenate(xparts, axis=0)
        d //= 2
    return k, x


def _repack(h, pw, side):
    if pw == 0:
        if side == 0:
            return h
        hi = jax.lax.bitcast_convert_type(h, jnp.int32) + 1
        return jax.lax.bitcast_convert_type(hi, jnp.float32)
    low = (1 << pw) - 1
    hi = jax.lax.bitcast_convert_type(h, jnp.int32)
    hi = hi + (hi & low) + side
    return jax.lax.bitcast_convert_type(hi, jnp.float32)


def _merge_packed(a, pw, truncate):
    R, L, _ = a.shape
    A = _repack(a[:, :L // 2], pw, 0)
    Br = _rev0(_repack(a[:, L // 2:], pw, 1))
    if truncate:
        lo, _ = _bitonic_clean(jnp.minimum(A, Br), None)
        return lo
    v = jnp.concatenate([A, Br], axis=0)
    v, _ = _bitonic_clean(v, None)
    return v


def _merge_kv(k, x, truncate):
    R, L, _ = k.shape
    ka, xa = k[:, :L // 2], x[:, :L // 2]
    kb = _rev0(k[:, L // 2:])
    xb = _rev0(x[:, L // 2:])
    if truncate:
        klo, xlo, _, _ = _cmpex(ka, xa, kb, xb)
        return _bitonic_clean(klo, xlo)
    kc = jnp.concatenate([ka, kb], axis=0)
    xc = jnp.concatenate([xa, xb], axis=0)
    return _bitonic_clean(kc, xc)


_UNPACK_L = 16


def _batcher_pairs(n):
    pairs = []

    def merge(lo, m, r):
        step = r * 2
        if step < m:
            merge(lo, m, step)
            merge(lo + r, m, step)
            for i in range(lo + r, lo + m - r, step):
                pairs.append((i, i + r))
        else:
            pairs.append((lo, lo + r))

    def sort(lo, m):
        if m > 1:
            half = m // 2
            sort(lo, half)
            sort(lo + half, half)
            merge(lo, m, 1)

    sort(0, n)
    return pairs


def _sort_net_list(vals):
    for i, j in _batcher_pairs(len(vals)):
        mn = jnp.minimum(vals[i], vals[j])
        mx = jnp.maximum(vals[i], vals[j])
        vals[i], vals[j] = mn, mx
    return vals


def _bitonic_sort_r(a):
    R, L, T = a.shape
    CH = 8
    out_chunks = []
    for l0 in range(0, L, CH):
        vals = [a[r, l0:l0 + CH] for r in range(R)]
        vals = _sort_net_list(vals)
        out_chunks.append(jnp.concatenate([v[None] for v in vals], axis=0))
    return jnp.concatenate(out_chunks, axis=1)


def _knn_body(pfull_ref, ptile_ref, idx_ref):
    p_all = pfull_ref[...]
    p_til = ptile_ref[...]
    T = p_til.shape[0]
    sq = jnp.sum(p_all * p_all, axis=1)
    sqt = jnp.sum(p_til * p_til, axis=1)
    dot = jax.lax.dot_general(p_all, p_til, (((1,), (1,)), ((), ())),
                              preferred_element_type=jnp.float32)
    key = (sq[:, None] - 2.0 * dot) + sqt[None, :]
    key = jnp.maximum(key, 1e-30)
    kb = jax.lax.bitcast_convert_type(key, jnp.int32)
    kb = kb & jnp.int32(~0xFF)
    kb = kb.reshape(K, N // K, T)
    kb = kb + jax.lax.broadcasted_iota(jnp.int32, kb.shape, 0)
    a = jax.lax.bitcast_convert_type(kb, jnp.float32)

    a = _bitonic_sort_r(a)
    pw = 5
    while a.shape[1] > _UNPACK_L:
        a = _merge_packed(a, pw, truncate=True)
        pw += 1
    L = a.shape[1]
    ai = jax.lax.bitcast_convert_type(a, jnp.int32)
    m = ai & jnp.int32((1 << pw) - 1)
    l_iota = jax.lax.broadcasted_iota(jnp.int32, a.shape, 1)
    x = m * L + l_iota
    k = a
    while k.shape[1] > 1:
        k, x = _merge_kv(k, x, truncate=True)
    idx_ref[...] = x[:, 0, :]


def _knn(p_b):
    T = 256
    grid = (N // T,)
    return pl.pallas_call(
        _knn_body,
        grid=grid,
        in_specs=[
            pl.BlockSpec((N, 3), lambda i: (0, 0)),
            pl.BlockSpec((T, 3), lambda i: (i, 0)),
        ],
        out_specs=pl.BlockSpec((K, T), lambda i: (0, i)),
        out_shape=jax.ShapeDtypeStruct((K, N), jnp.int32),
    )(p_b, p_b)


_N_IDX = K * N
_SC_W = 32
_PER_W = _N_IDX // _SC_W
_CHUNK = 256
_ROW_W = 2 * D_HID


def _sc_gather_body(t_ref, idx_ref, out_ref, idx0, idx1, rows0, rows1,
                    gsem0, gsem1, wsem0, wsem1):
    wid = jax.lax.axis_index("c") * 16 + jax.lax.axis_index("s")
    base = wid * _PER_W
    idx_v = (idx0, idx1)
    rows_v = (rows0, rows1)
    gsem = (gsem0, gsem1)
    wsem = (wsem0, wsem1)
    n_it = _PER_W // _CHUNK
    writes = [None] * n_it
    for i in range(n_it):
        bi = i & 1
        if i >= 2:
            writes[i - 2].wait()
        off = base + i * _CHUNK
        pltpu.sync_copy(idx_ref.at[pl.ds(off, _CHUNK)], idx_v[bi])
        pltpu.async_copy(t_ref.at[idx_v[bi]], rows_v[bi], gsem[bi]).wait()
        writes[i] = pltpu.async_copy(rows_v[bi], out_ref.at[pl.ds(off, _CHUNK)],
                                     wsem[bi])
    writes[n_it - 2].wait()
    writes[n_it - 1].wait()


def _sc_gather(t_flat, idx_flat):
    mesh = plsc.VectorSubcoreMesh(core_axis_name="c", subcore_axis_name="s")
    fn = functools.partial(
        pl.kernel,
        mesh=mesh,
        out_type=jax.ShapeDtypeStruct((_N_IDX, _ROW_W), jnp.float32),
        scratch_types=[
            pltpu.VMEM((_CHUNK,), jnp.int32),
            pltpu.VMEM((_CHUNK,), jnp.int32),
            pltpu.VMEM((_CHUNK, _ROW_W), jnp.float32),
            pltpu.VMEM((_CHUNK, _ROW_W), jnp.float32),
            pltpu.SemaphoreType.DMA,
            pltpu.SemaphoreType.DMA,
            pltpu.SemaphoreType.DMA,
            pltpu.SemaphoreType.DMA,
        ],
    )(_sc_gather_body)
    return fn(t_flat, idx_flat)


def kernel(f, p, W_sc, b_sc, W_pre, b_pre, W_p2f, b_p2f, W_mlp, b_mlp,
           W_pst, b_pst):
    A = W_p2f[0:3] - W_p2f[6:9]
    Bm = W_p2f[3:6] + W_p2f[6:9]
    W_pab = jnp.concatenate([A, Bm], axis=1)

    tc_pack, res = _prologue(f, p, W_pre, b_pre, W_sc, b_sc, W_pab, b_p2f)

    idxs = [_knn(p[b]) for b in range(B)]
    tgs = [_sc_gather(tc_pack[b], idxs[b].reshape(-1)) for b in range(B)]
    outs = [_tail(tgs[b].reshape(K, N, _ROW_W), tc_pack[b], res[b],
                  W_mlp, b_mlp, W_pst, b_pst) for b in range(B)]
    return (jnp.stack(outs), p)


# speedup vs baseline: 1.8338x; 1.0640x over previous
import functools
import jax
import jax.numpy as jnp
from jax.experimental import pallas as pl
from jax.experimental.pallas import tpu as pltpu
from jax.experimental.pallas import tpu_sc as plsc

B, N, D_IN, D_OUT, D_HID = 4, 4096, 128, 256, 64
K = 32


def _prologue_body(f_ref, p_ref, wpre_ref, bpre_ref, wsc_ref, bsc_ref,
                   wpab_ref, bp2f_ref, tc_ref, res_ref):
    f = f_ref[0]
    p = p_ref[0]
    h = f @ wpre_ref[...] + bpre_ref[...]
    h = jnp.maximum(h, 0.01 * h)
    pa = p @ wpab_ref[...]
    t = h + pa[:, D_HID:]
    c = pa[:, :D_HID] - h + bp2f_ref[...]
    tc_ref[0] = jnp.concatenate([t, c], axis=1)
    res_ref[0] = f @ wsc_ref[...] + bsc_ref[...]


def _prologue(f, p, W_pre, b_pre, W_sc, b_sc, W_pab, b_p2f):
    T = 1024
    grid = (B, N // T)
    return pl.pallas_call(
        _prologue_body,
        grid=grid,
        in_specs=[
            pl.BlockSpec((1, T, D_IN), lambda b, i: (b, i, 0)),
            pl.BlockSpec((1, T, 3), lambda b, i: (b, i, 0)),
            pl.BlockSpec((D_IN, D_HID), lambda b, i: (0, 0)),
            pl.BlockSpec((D_HID,), lambda b, i: (0,)),
            pl.BlockSpec((D_IN, D_OUT), lambda b, i: (0, 0)),
            pl.BlockSpec((D_OUT,), lambda b, i: (0,)),
            pl.BlockSpec((3, 2 * D_HID), lambda b, i: (0, 0)),
            pl.BlockSpec((D_HID,), lambda b, i: (0,)),
        ],
        out_specs=[
            pl.BlockSpec((1, T, 2 * D_HID), lambda b, i: (b, i, 0)),
            pl.BlockSpec((1, T, D_OUT), lambda b, i: (b, i, 0)),
        ],
        out_shape=[
            jax.ShapeDtypeStruct((B, N, 2 * D_HID), jnp.float32),
            jax.ShapeDtypeStruct((B, N, D_OUT), jnp.float32),
        ],
    )(f, p, W_pre, b_pre, W_sc, b_sc, W_pab, b_p2f)


def _tail_body(tg_ref, c_ref, res_ref, wmlp_ref, bmlp_ref, wpst_ref,
               bpst_ref, out_ref):
    Trows = c_ref.shape[0]
    tg = tg_ref[:, :, :D_HID]
    c = c_ref[:, D_HID:]
    g = tg + c[None, :, :]
    g = jnp.maximum(g, 0.01 * g)
    z = g.reshape(K * Trows, D_HID) @ wmlp_ref[...] + bmlp_ref[...]
    z = jnp.maximum(z, 0.01 * z)
    m = jnp.max(z.reshape(K, Trows, D_HID), axis=0)
    o = m @ wpst_ref[...] + bpst_ref[...] + res_ref[...]
    out_ref[...] = jnp.maximum(o, 0.01 * o)


def _tail(tg_b, tc_b, res_b, W_mlp, b_mlp, W_pst, b_pst):
    T = 512
    grid = (N // T,)
    return pl.pallas_call(
        _tail_body,
        grid=grid,
        in_specs=[
            pl.BlockSpec((K, T, 2 * D_HID), lambda i: (0, i, 0)),
            pl.BlockSpec((T, 2 * D_HID), lambda i: (i, 0)),
            pl.BlockSpec((T, D_OUT), lambda i: (i, 0)),
            pl.BlockSpec((D_HID, D_HID), lambda i: (0, 0)),
            pl.BlockSpec((D_HID,), lambda i: (0,)),
            pl.BlockSpec((D_HID, D_OUT), lambda i: (0, 0)),
            pl.BlockSpec((D_OUT,), lambda i: (0,)),
        ],
        out_specs=pl.BlockSpec((T, D_OUT), lambda i: (i, 0)),
        out_shape=jax.ShapeDtypeStruct((N, D_OUT), jnp.float32),
    )(tg_b, tc_b, res_b, W_mlp, b_mlp, W_pst, b_pst)


def _rev0(a):
    if a.shape[0] == 1:
        return a
    return jnp.concatenate([a[i:i + 1] for i in range(a.shape[0] - 1, -1, -1)],
                           axis=0)


def _cmpex(ka, xa, kb, xb):
    m = kb < ka
    klo = jnp.where(m, kb, ka)
    khi = jnp.where(m, ka, kb)
    xlo = jnp.where(m, xb, xa)
    xhi = jnp.where(m, xa, xb)
    return klo, xlo, khi, xhi


def _bitonic_clean(k, x):
    R = k.shape[0]
    d = R // 2
    while d >= 1:
        kparts, xparts = [], []
        for b in range(0, R, 2 * d):
            ka, kb = k[b:b + d], k[b + d:b + 2 * d]
            if x is None:
                kparts += [jnp.minimum(ka, kb), jnp.maximum(ka, kb)]
            else:
                klo, xlo, khi, xhi = _cmpex(ka, x[b:b + d], kb,
                                            x[b + d:b + 2 * d])
                kparts += [klo, khi]
                xparts += [xlo, xhi]
        k = jnp.concatenate(kparts, axis=0)
        if x is not None:
            x = jnp.concatenate(xparts, axis=0)
        d //= 2
    return k, x


def _repack(h, pw, side):
    if pw == 0:
        if side == 0:
            return h
        hi = jax.lax.bitcast_convert_type(h, jnp.int32) + 1
        return jax.lax.bitcast_convert_type(hi, jnp.float32)
    low = (1 << pw) - 1
    hi = jax.lax.bitcast_convert_type(h, jnp.int32)
    hi = hi + (hi & low) + side
    return jax.lax.bitcast_convert_type(hi, jnp.float32)


def _merge_packed(a, pw, truncate):
    R, L, _ = a.shape
    A = _repack(a[:, :L // 2], pw, 0)
    Br = _rev0(_repack(a[:, L // 2:], pw, 1))
    if truncate:
        lo, _ = _bitonic_clean(jnp.minimum(A, Br), None)
        return lo
    v = jnp.concatenate([A, Br], axis=0)
    v, _ = _bitonic_clean(v, None)
    return v


def _merge_kv(k, x, truncate):
    R, L, _ = k.shape
    ka, xa = k[:, :L // 2], x[:, :L // 2]
    kb = _rev0(k[:, L // 2:])
    xb = _rev0(x[:, L // 2:])
    if truncate:
        klo, xlo, _, _ = _cmpex(ka, xa, kb, xb)
        return _bitonic_clean(klo, xlo)
    kc = jnp.concatenate([ka, kb], axis=0)
    xc = jnp.concatenate([xa, xb], axis=0)
    return _bitonic_clean(kc, xc)


_UNPACK_L = 16


def _batcher_pairs(n):
    pairs = []

    def merge(lo, m, r):
        step = r * 2
        if step < m:
            merge(lo, m, step)
            merge(lo + r, m, step)
            for i in range(lo + r, lo + m - r, step):
                pairs.append((i, i + r))
        else:
            pairs.append((lo, lo + r))

    def sort(lo, m):
        if m > 1:
            half = m // 2
            sort(lo, half)
            sort(lo + half, half)
            merge(lo, m, 1)

    sort(0, n)
    return pairs


def _sort_net_list(vals):
    for i, j in _batcher_pairs(len(vals)):
        mn = jnp.minimum(vals[i], vals[j])
        mx = jnp.maximum(vals[i], vals[j])
        vals[i], vals[j] = mn, mx
    return vals


def _bitonic_sort_r(a):
    R, L, T = a.shape
    CH = 8
    out_chunks = []
    for l0 in range(0, L, CH):
        vals = [a[r, l0:l0 + CH] for r in range(R)]
        vals = _sort_net_list(vals)
        out_chunks.append(jnp.concatenate([v[None] for v in vals], axis=0))
    return jnp.concatenate(out_chunks, axis=1)


def _knn_body(pfull_ref, ptile_ref, idx_ref):
    p_all = pfull_ref[...]
    p_til = ptile_ref[...]
    T = p_til.shape[0]
    sq = jnp.sum(p_all * p_all, axis=1)
    sqt = jnp.sum(p_til * p_til, axis=1)
    dot = jax.lax.dot_general(p_all, p_til, (((1,), (1,)), ((), ())),
                              preferred_element_type=jnp.float32)
    key = (sq[:, None] - 2.0 * dot) + sqt[None, :]
    key = jnp.maximum(key, 1e-30)
    kb = jax.lax.bitcast_convert_type(key, jnp.int32)
    kb = kb & jnp.int32(~0xFF)
    kb = kb.reshape(K, N // K, T)
    kb = kb + jax.lax.broadcasted_iota(jnp.int32, kb.shape, 0)
    a = jax.lax.bitcast_convert_type(kb, jnp.float32)

    a = _bitonic_sort_r(a)
    pw = 5
    while a.shape[1] > _UNPACK_L:
        a = _merge_packed(a, pw, truncate=True)
        pw += 1
    L = a.shape[1]
    ai = jax.lax.bitcast_convert_type(a, jnp.int32)
    m = ai & jnp.int32((1 << pw) - 1)
    l_iota = jax.lax.broadcasted_iota(jnp.int32, a.shape, 1)
    x = m * L + l_iota
    k = a
    while k.shape[1] > 1:
        k, x = _merge_kv(k, x, truncate=True)
    idx_ref[...] = x[:, 0, :]


def _knn(p_b):
    T = 512
    grid = (N // T,)
    return pl.pallas_call(
        _knn_body,
        grid=grid,
        in_specs=[
            pl.BlockSpec((N, 3), lambda i: (0, 0)),
            pl.BlockSpec((T, 3), lambda i: (i, 0)),
        ],
        out_specs=pl.BlockSpec((K, T), lambda i: (0, i)),
        out_shape=jax.ShapeDtypeStruct((K, N), jnp.int32),
    )(p_b, p_b)


_N_IDX = K * N
_SC_W = 32
_PER_W = _N_IDX // _SC_W
_CHUNK = 256
_ROW_W = 2 * D_HID


def _sc_gather_body(t_ref, idx_ref, out_ref, idx0, idx1, rows0, rows1,
                    gsem0, gsem1, wsem0, wsem1):
    wid = jax.lax.axis_index("c") * 16 + jax.lax.axis_index("s")
    base = wid * _PER_W
    idx_v = (idx0, idx1)
    rows_v = (rows0, rows1)
    gsem = (gsem0, gsem1)
    wsem = (wsem0, wsem1)
    n_it = _PER_W // _CHUNK
    writes = [None] * n_it
    for i in range(n_it):
        bi = i & 1
        if i >= 2:
            writes[i - 2].wait()
        off = base + i * _CHUNK
        pltpu.sync_copy(idx_ref.at[pl.ds(off, _CHUNK)], idx_v[bi])
        pltpu.async_copy(t_ref.at[idx_v[bi]], rows_v[bi], gsem[bi]).wait()
        writes[i] = pltpu.async_copy(rows_v[bi], out_ref.at[pl.ds(off, _CHUNK)],
                                     wsem[bi])
    writes[n_it - 2].wait()
    writes[n_it - 1].wait()


def _sc_gather(t_flat, idx_flat):
    mesh = plsc.VectorSubcoreMesh(core_axis_name="c", subcore_axis_name="s")
    fn = functools.partial(
        pl.kernel,
        mesh=mesh,
        out_type=jax.ShapeDtypeStruct((_N_IDX, _ROW_W), jnp.float32),
        scratch_types=[
            pltpu.VMEM((_CHUNK,), jnp.int32),
            pltpu.VMEM((_CHUNK,), jnp.int32),
            pltpu.VMEM((_CHUNK, _ROW_W), jnp.float32),
            pltpu.VMEM((_CHUNK, _ROW_W), jnp.float32),
            pltpu.SemaphoreType.DMA,
            pltpu.SemaphoreType.DMA,
            pltpu.SemaphoreType.DMA,
            pltpu.SemaphoreType.DMA,
        ],
    )(_sc_gather_body)
    return fn(t_flat, idx_flat)


def kernel(f, p, W_sc, b_sc, W_pre, b_pre, W_p2f, b_p2f, W_mlp, b_mlp,
           W_pst, b_pst):
    A = W_p2f[0:3] - W_p2f[6:9]
    Bm = W_p2f[3:6] + W_p2f[6:9]
    W_pab = jnp.concatenate([A, Bm], axis=1)

    tc_pack, res = _prologue(f, p, W_pre, b_pre, W_sc, b_sc, W_pab, b_p2f)

    idxs = [_knn(p[b]) for b in range(B)]
    tgs = [_sc_gather(tc_pack[b], idxs[b].reshape(-1)) for b in range(B)]
    outs = [_tail(tgs[b].reshape(K, N, _ROW_W), tc_pack[b], res[b],
                  W_mlp, b_mlp, W_pst, b_pst) for b in range(B)]
    return (jnp.stack(outs), p)
